# speedup vs baseline: 1.0532x; 1.0532x over previous
import functools

import jax
import jax.numpy as jnp
from jax import lax
from jax.experimental import pallas as pl
from jax.experimental.pallas import tpu as pltpu
from jax.experimental.pallas import tpu_sc as plsc

NC = 2
NS = 16
NW = NC * NS
L = 16

B = 16384
BPW = B // NW
CH = 128
NCH = BPW // CH

_mesh = plsc.VectorSubcoreMesh(core_axis_name="c", subcore_axis_name="s")


@functools.partial(
    pl.kernel,
    mesh=_mesh,
    out_type=jax.ShapeDtypeStruct((B,), jnp.float32),
    scratch_types=[
        pltpu.VMEM((BPW,), jnp.int32),
        pltpu.VMEM((BPW,), jnp.int32),
        pltpu.VMEM((NCH, CH), jnp.int32),
        pltpu.VMEM((NCH, CH), jnp.float32),
        pltpu.SemaphoreType.DMA,
    ],
)
def _lookup(x0_hbm, x1_hbm, table_hbm, out_hbm, x0_v, x1_v, idx_v, val_v, sem):
    wid = lax.axis_index("s") * NC + lax.axis_index("c")
    base = wid * BPW
    pltpu.sync_copy(x0_hbm.at[pl.ds(base, BPW)], x0_v)
    pltpu.sync_copy(x1_hbm.at[pl.ds(base, BPW)], x1_v)
    for j in range(NCH):
        row = idx_v.at[j]
        for i in range(CH // L):
            s = pl.ds(j * CH + i * L, L)
            row[pl.ds(i * L, L)] = x0_v[s] * 1000 + x1_v[s]
    copies = [
        pltpu.async_copy(table_hbm.at[idx_v.at[j]], val_v.at[j], sem)
        for j in range(NCH)
    ]
    for c in copies:
        c.wait()
    for j in range(NCH):
        pltpu.sync_copy(val_v.at[j], out_hbm.at[pl.ds(base + j * CH, CH)])


def kernel(xs, param_vec):
    x0 = xs[:, 0].astype(jnp.int32)
    x1 = xs[:, 1].astype(jnp.int32)
    return _lookup(x0, x1, param_vec)
